# baseline (speedup 1.0000x reference)
import functools

import jax
import jax.numpy as jnp
from jax import lax
from jax.experimental import pallas as pl
from jax.experimental.pallas import tpu as pltpu
from jax.experimental.pallas import tpu_sc as plsc

NUM_SEGMENTS = 16
ROWS_PER_SEG = 2048
COLS = 1024

NC = 2
NS = 16
ROWS_PER_W = ROWS_PER_SEG // 2
NV = COLS // 16
RC = 32
NCH = ROWS_PER_W // RC
UNROLL = 4


@functools.partial(
    pl.kernel,
    out_type=jax.ShapeDtypeStruct((NUM_SEGMENTS, COLS), jnp.float32),
    mesh=plsc.VectorSubcoreMesh(core_axis_name="c", subcore_axis_name="s"),
    scratch_types=[
        pltpu.VMEM((RC, COLS), jnp.float32),
        pltpu.VMEM((RC, COLS), jnp.float32),
        pltpu.VMEM((COLS,), jnp.float32),
        pltpu.VMEM((COLS,), jnp.float32),
        pltpu.VMEM_SHARED((NS, COLS), jnp.float32),
        pltpu.SemaphoreType.DMA,
        pltpu.SemaphoreType.DMA,
        pltpu.SemaphoreType.DMA,
    ],
)
def _sc_segmean(inp_hbm, out_hbm, buf0, buf1, acc, pbuf, shared, sem0, sem1,
                sem2):
    c = lax.axis_index("c")
    s = lax.axis_index("s")
    seg = c * 8 + s // 2
    half = s % 2
    row0 = seg * ROWS_PER_SEG + half * ROWS_PER_W

    bufs = (buf0, buf1)
    sems = (sem0, sem1)

    def start(k, b):
        pltpu.make_async_copy(
            inp_hbm.at[pl.ds(row0 + k * RC, RC), :],
            bufs[b],
            sems[b],
        ).start()

    def wait(b):
        pltpu.make_async_copy(
            inp_hbm.at[pl.ds(row0, RC), :],
            bufs[b],
            sems[b],
        ).wait()

    def accum(buf):
        def body(i, carry):
            r = i * UNROLL
            for j in range(NV):
                cc = j * 16
                v = acc[pl.ds(cc, 16)]
                for u in range(UNROLL):
                    v = v + buf[r + u, pl.ds(cc, 16)]
                acc[pl.ds(cc, 16)] = v
            return carry
        lax.fori_loop(0, RC // UNROLL, body, 0)

    start(0, 0)
    start(1, 1)
    zero = jnp.zeros((16,), jnp.float32)
    for j in range(NV):
        acc[pl.ds(j * 16, 16)] = zero

    def ring(i, carry):
        for b in range(2):
            k = i * 2 + b
            wait(b)
            accum(bufs[b])
            start(k + 2, b)
        return carry
    lax.fori_loop(0, (NCH - 2) // 2, ring, 0)

    wait(0)
    accum(buf0)
    wait(1)
    accum(buf1)

    pltpu.sync_copy(acc, shared.at[s])
    plsc.subcore_barrier()

    @pl.when(half == 0)
    def _combine():
        pltpu.sync_copy(shared.at[s + 1], pbuf)
        scale = jnp.float32(1.0 / ROWS_PER_SEG)
        for j in range(NV):
            cc = j * 16
            acc[pl.ds(cc, 16)] = (acc[pl.ds(cc, 16)] + pbuf[pl.ds(cc, 16)]) * scale
        pltpu.make_async_copy(acc, out_hbm.at[seg], sem2).start()
        pltpu.make_async_copy(acc, out_hbm.at[seg], sem2).wait()


def kernel(inp):
    return _sc_segmean(inp)


# speedup vs baseline: 2.0870x; 2.0870x over previous
import functools

import jax
import jax.numpy as jnp
from jax import lax
from jax.experimental import pallas as pl
from jax.experimental.pallas import tpu as pltpu
from jax.experimental.pallas import tpu_sc as plsc

NUM_SEGMENTS = 16
ROWS_PER_SEG = 2048
COLS = 1024

NC = 2
NS = 16
ROWS_PER_W = ROWS_PER_SEG // 2
NV = COLS // 16
RC = 32
NCH = ROWS_PER_W // RC
UNROLL = 4


@functools.partial(
    pl.kernel,
    out_type=jax.ShapeDtypeStruct((NUM_SEGMENTS, COLS), jnp.float32),
    mesh=plsc.VectorSubcoreMesh(core_axis_name="c", subcore_axis_name="s"),
    scratch_types=[
        pltpu.VMEM((RC, COLS), jnp.float32),
        pltpu.VMEM((RC, COLS), jnp.float32),
        pltpu.VMEM((COLS,), jnp.float32),
        pltpu.VMEM((COLS,), jnp.float32),
        pltpu.VMEM_SHARED((NS, COLS), jnp.float32),
        pltpu.SemaphoreType.DMA,
        pltpu.SemaphoreType.DMA,
        pltpu.SemaphoreType.DMA,
    ],
)
def _sc_segmean(inp_hbm, out_hbm, buf0, buf1, acc, pbuf, shared, sem0, sem1,
                sem2):
    c = lax.axis_index("c")
    s = lax.axis_index("s")
    seg = c * 8 + s // 2
    half = s % 2
    row0 = seg * ROWS_PER_SEG + half * ROWS_PER_W

    bufs = (buf0, buf1)
    sems = (sem0, sem1)

    def start(k, b):
        pltpu.make_async_copy(
            inp_hbm.at[pl.ds(row0 + k * RC, RC), :],
            bufs[b],
            sems[b],
        ).start()

    def wait(b):
        pltpu.make_async_copy(
            inp_hbm.at[pl.ds(row0, RC), :],
            bufs[b],
            sems[b],
        ).wait()

    def accum(buf):
        @plsc.parallel_loop(0, NV, unroll=2)
        def _jbody(j):
            cc = j * 16
            v = acc[pl.ds(cc, 16)]
            parts = []
            for g in range(RC // 8):
                t = buf[g * 8, pl.ds(cc, 16)]
                for r in range(g * 8 + 1, g * 8 + 8):
                    t = t + buf[r, pl.ds(cc, 16)]
                parts.append(t)
            while len(parts) > 1:
                parts = [a + b for a, b in zip(parts[::2], parts[1::2])]
            acc[pl.ds(cc, 16)] = v + parts[0]

    start(0, 0)
    start(1, 1)
    zero = jnp.zeros((16,), jnp.float32)
    for j in range(NV):
        acc[pl.ds(j * 16, 16)] = zero

    def ring(i, carry):
        for b in range(2):
            k = i * 2 + b
            wait(b)
            accum(bufs[b])
            start(k + 2, b)
        return carry
    lax.fori_loop(0, (NCH - 2) // 2, ring, 0)

    wait(0)
    accum(buf0)
    wait(1)
    accum(buf1)

    pltpu.sync_copy(acc, shared.at[s])
    plsc.subcore_barrier()

    @pl.when(half == 0)
    def _combine():
        pltpu.sync_copy(shared.at[s + 1], pbuf)
        scale = jnp.float32(1.0 / ROWS_PER_SEG)
        for j in range(NV):
            cc = j * 16
            acc[pl.ds(cc, 16)] = (acc[pl.ds(cc, 16)] + pbuf[pl.ds(cc, 16)]) * scale
        pltpu.make_async_copy(acc, out_hbm.at[seg], sem2).start()
        pltpu.make_async_copy(acc, out_hbm.at[seg], sem2).wait()


def kernel(inp):
    return _sc_segmean(inp)
